# Initial kernel scaffold; baseline (speedup 1.0000x reference)
import jax
import jax.numpy as jnp
from jax.experimental import pallas as pl


def kernel(data, segment_ids):
    raise NotImplementedError("write your pallas kernel here")


# speedup vs baseline: 4.6639x; 4.6639x over previous
import jax
import jax.numpy as jnp
from jax import lax
from jax.experimental import pallas as pl
from jax.experimental.pallas import tpu as pltpu
from jax.experimental.pallas import tpu_sc as plsc

N = 320000
D = 128
S = 10000
NC = 2
NS = 16
NW = NC * NS
R = N // NW
CH = 128
NFULL = R // CH
TAIL = R - NFULL * CH
SPW = 624
SREM = S - NS * SPW
CW = 16

_MESH = plsc.VectorSubcoreMesh(
    core_axis_name="c", subcore_axis_name="s", num_cores=NC, num_subcores=NS
)


def _zero_slices(src, dst, sid):
    for o, w in [(0, CH), (CH, CH), (2 * CH, CH), (3 * CH, CH), (4 * CH, 112)]:
        pltpu.sync_copy(src.at[pl.ds(0, w)], dst.at[pl.ds(sid * SPW + o, w)])

    @pl.when(sid == NS - 1)
    def _zero_rem():
        pltpu.sync_copy(src.at[pl.ds(0, SREM)], dst.at[pl.ds(NS * SPW, SREM)])


def _sum_body(data_hbm, seg_hbm, acc_out, dbuf, ibuf, itail, acc_sh):
    cid = lax.axis_index("c")
    sid = lax.axis_index("s")
    base = (cid * NS + sid) * R

    zeros16 = jnp.zeros((16,), jnp.float32)

    def init_row(r, carry):
        for k in range(D // 16):
            dbuf[r, pl.ds(k * 16, 16)] = zeros16
        return carry

    lax.fori_loop(0, CH, init_row, 0)
    _zero_slices(dbuf, acc_sh, sid)
    plsc.subcore_barrier()

    def chunk(i, carry):
        off = base + i * CH
        pltpu.sync_copy(data_hbm.at[pl.ds(off, CH)], dbuf)
        pltpu.sync_copy(seg_hbm.at[pl.ds(off, CH)], ibuf)
        pltpu.sync_copy(dbuf, acc_sh.at[ibuf], add=True)
        return carry

    lax.fori_loop(0, NFULL, chunk, 0)

    toff = base + NFULL * CH
    pltpu.sync_copy(data_hbm.at[pl.ds(toff, TAIL)], dbuf.at[pl.ds(0, TAIL)])
    pltpu.sync_copy(seg_hbm.at[pl.ds(toff, TAIL)], itail)
    pltpu.sync_copy(dbuf.at[pl.ds(0, TAIL)], acc_sh.at[itail], add=True)

    plsc.subcore_barrier()

    r0 = sid * SPW
    pltpu.sync_copy(acc_sh.at[pl.ds(r0, SPW)], acc_out.at[cid, pl.ds(r0, SPW)])

    @pl.when(sid == NS - 1)
    def _read_rem():
        b = NS * SPW
        pltpu.sync_copy(acc_sh.at[pl.ds(b, SREM)], acc_out.at[cid, pl.ds(b, SREM)])


def _cnt_body(seg_hbm, aux_hbm, cnt_out, ibuf, itail, obuf, zbuf, cnt_sh):
    cid = lax.axis_index("c")
    sid = lax.axis_index("s")
    base = (cid * NS + sid) * R

    pltpu.sync_copy(aux_hbm.at[0], obuf)
    pltpu.sync_copy(aux_hbm.at[1], zbuf)
    _zero_slices(zbuf, cnt_sh, sid)
    plsc.subcore_barrier()

    def chunk(i, carry):
        pltpu.sync_copy(seg_hbm.at[pl.ds(base + i * CH, CH)], ibuf)
        pltpu.sync_copy(obuf, cnt_sh.at[ibuf], add=True)
        return carry

    lax.fori_loop(0, NFULL, chunk, 0)

    pltpu.sync_copy(seg_hbm.at[pl.ds(base + NFULL * CH, TAIL)], itail)
    pltpu.sync_copy(obuf.at[pl.ds(0, TAIL)], cnt_sh.at[itail], add=True)

    plsc.subcore_barrier()

    r0 = sid * SPW
    pltpu.sync_copy(cnt_sh.at[pl.ds(r0, SPW)], cnt_out.at[cid, pl.ds(r0, SPW)])

    @pl.when(sid == NS - 1)
    def _read_rem():
        b = NS * SPW
        pltpu.sync_copy(cnt_sh.at[pl.ds(b, SREM)], cnt_out.at[cid, pl.ds(b, SREM)])


def _sc_sums(data, seg):
    return pl.kernel(
        _sum_body,
        out_type=jax.ShapeDtypeStruct((NC, S, D), jnp.float32),
        mesh=_MESH,
        scratch_types=[
            pltpu.VMEM((CH, D), jnp.float32),
            pltpu.VMEM((CH,), jnp.int32),
            pltpu.VMEM((TAIL,), jnp.int32),
            pltpu.VMEM_SHARED((S, D), jnp.float32),
        ],
    )(data, seg)


def _sc_counts(seg, aux):
    return pl.kernel(
        _cnt_body,
        out_type=jax.ShapeDtypeStruct((NC, S, CW), jnp.float32),
        mesh=_MESH,
        scratch_types=[
            pltpu.VMEM((CH,), jnp.int32),
            pltpu.VMEM((TAIL,), jnp.int32),
            pltpu.VMEM((CH, CW), jnp.float32),
            pltpu.VMEM((CH, CW), jnp.float32),
            pltpu.VMEM_SHARED((S, CW), jnp.float32),
        ],
    )(seg, aux)


def _combine_body(acc_ref, cnt_ref, out_ref):
    sums = acc_ref[0] + acc_ref[1]
    counts = cnt_ref[0, :, 0:1] + cnt_ref[1, :, 0:1]
    out_ref[...] = sums / jnp.maximum(counts, 1.0)


@jax.jit
def kernel(data, segment_ids):
    seg = segment_ids.astype(jnp.int32)
    aux = jnp.concatenate([jnp.ones((1, CH, CW), jnp.float32),
                           jnp.zeros((1, CH, CW), jnp.float32)])
    acc = _sc_sums(data, seg)
    cnt = _sc_counts(seg, aux)
    return pl.pallas_call(
        _combine_body,
        out_shape=jax.ShapeDtypeStruct((S, D), jnp.float32),
    )(acc, cnt)


# speedup vs baseline: 7.7178x; 1.6548x over previous
import jax
import jax.numpy as jnp
from jax import lax
from jax.experimental import pallas as pl
from jax.experimental.pallas import tpu as pltpu
from jax.experimental.pallas import tpu_sc as plsc

N = 320000
D = 128
S = 10000
NC = 2
NS = 16
NW = NC * NS
R = N // NW
CH = 128
NFULL = R // CH
TAIL = R - NFULL * CH
NB = 2
NMACRO = NFULL // NB
SPW = 624
SREM = S - NS * SPW
CW = 16

_MESH = plsc.VectorSubcoreMesh(
    core_axis_name="c", subcore_axis_name="s", num_cores=NC, num_subcores=NS
)


def _zero_slices(src, dst, sid):
    for o, w in [(0, CH), (CH, CH), (2 * CH, CH), (3 * CH, CH), (4 * CH, 112)]:
        pltpu.sync_copy(src.at[pl.ds(0, w)], dst.at[pl.ds(sid * SPW + o, w)])

    @pl.when(sid == NS - 1)
    def _zero_rem():
        pltpu.sync_copy(src.at[pl.ds(0, SREM)], dst.at[pl.ds(NS * SPW, SREM)])


def _sum_body(data_hbm, seg_hbm, acc_out,
              dbuf, ibuf, itail, dsem, isem, ssem, acc_sh):
    cid = lax.axis_index("c")
    sid = lax.axis_index("s")
    base = (cid * NS + sid) * R

    zeros16 = jnp.zeros((16,), jnp.float32)

    def init_row(r, carry):
        for k in range(D // 16):
            dbuf[0, r, pl.ds(k * 16, 16)] = zeros16
        return carry

    lax.fori_loop(0, CH, init_row, 0)
    _zero_slices(dbuf.at[0], acc_sh, sid)
    plsc.subcore_barrier()

    for b in range(NB):
        off = base + b * CH
        pltpu.async_copy(data_hbm.at[pl.ds(off, CH)], dbuf.at[b], dsem.at[b])
        pltpu.async_copy(seg_hbm.at[pl.ds(off, CH)], ibuf.at[b], isem.at[b])

    def macro(m, carry):
        for b in range(NB):
            off = base + (m * NB + b) * CH
            pltpu.make_async_copy(data_hbm.at[pl.ds(off, CH)],
                                  dbuf.at[b], dsem.at[b]).wait()
            pltpu.make_async_copy(seg_hbm.at[pl.ds(off, CH)],
                                  ibuf.at[b], isem.at[b]).wait()
            pltpu.async_copy(dbuf.at[b], acc_sh.at[ibuf.at[b]], ssem.at[b],
                             add=True).wait()

            @pl.when(m < NMACRO - 1)
            def _refill():
                noff = off + NB * CH
                pltpu.async_copy(data_hbm.at[pl.ds(noff, CH)],
                                 dbuf.at[b], dsem.at[b])
                pltpu.async_copy(seg_hbm.at[pl.ds(noff, CH)],
                                 ibuf.at[b], isem.at[b])
        return carry

    lax.fori_loop(0, NMACRO, macro, 0)

    toff = base + NFULL * CH
    pltpu.sync_copy(data_hbm.at[pl.ds(toff, TAIL)], dbuf.at[0, pl.ds(0, TAIL)])
    pltpu.sync_copy(seg_hbm.at[pl.ds(toff, TAIL)], itail)
    pltpu.sync_copy(dbuf.at[0, pl.ds(0, TAIL)], acc_sh.at[itail], add=True)

    plsc.subcore_barrier()

    r0 = sid * SPW
    pltpu.sync_copy(acc_sh.at[pl.ds(r0, SPW)], acc_out.at[cid, pl.ds(r0, SPW)])

    @pl.when(sid == NS - 1)
    def _read_rem():
        b = NS * SPW
        pltpu.sync_copy(acc_sh.at[pl.ds(b, SREM)], acc_out.at[cid, pl.ds(b, SREM)])


def _cnt_body(seg_hbm, aux_hbm, cnt_out, ibuf, itail, obuf, zbuf,
              isem, ssem, cnt_sh):
    cid = lax.axis_index("c")
    sid = lax.axis_index("s")
    base = (cid * NS + sid) * R

    pltpu.sync_copy(aux_hbm.at[0], obuf)
    pltpu.sync_copy(aux_hbm.at[1], zbuf)
    _zero_slices(zbuf, cnt_sh, sid)
    plsc.subcore_barrier()

    for b in range(NB):
        pltpu.async_copy(seg_hbm.at[pl.ds(base + b * CH, CH)],
                         ibuf.at[b], isem.at[b])

    def macro(m, carry):
        for b in range(NB):
            off = base + (m * NB + b) * CH
            pltpu.make_async_copy(seg_hbm.at[pl.ds(off, CH)],
                                  ibuf.at[b], isem.at[b]).wait()
            pltpu.async_copy(obuf, cnt_sh.at[ibuf.at[b]], ssem.at[b],
                             add=True).wait()

            @pl.when(m < NMACRO - 1)
            def _refill():
                pltpu.async_copy(seg_hbm.at[pl.ds(off + NB * CH, CH)],
                                 ibuf.at[b], isem.at[b])
        return carry

    lax.fori_loop(0, NMACRO, macro, 0)

    pltpu.sync_copy(seg_hbm.at[pl.ds(base + NFULL * CH, TAIL)], itail)
    pltpu.sync_copy(obuf.at[pl.ds(0, TAIL)], cnt_sh.at[itail], add=True)

    plsc.subcore_barrier()

    r0 = sid * SPW
    pltpu.sync_copy(cnt_sh.at[pl.ds(r0, SPW)], cnt_out.at[cid, pl.ds(r0, SPW)])

    @pl.when(sid == NS - 1)
    def _read_rem():
        b = NS * SPW
        pltpu.sync_copy(cnt_sh.at[pl.ds(b, SREM)], cnt_out.at[cid, pl.ds(b, SREM)])


def _sc_sums(data, seg):
    return pl.kernel(
        _sum_body,
        out_type=jax.ShapeDtypeStruct((NC, S, D), jnp.float32),
        mesh=_MESH,
        scratch_types=[
            pltpu.VMEM((NB, CH, D), jnp.float32),
            pltpu.VMEM((NB, CH), jnp.int32),
            pltpu.VMEM((TAIL,), jnp.int32),
            pltpu.SemaphoreType.DMA((NB,)),
            pltpu.SemaphoreType.DMA((NB,)),
            pltpu.SemaphoreType.DMA((NB,)),
            pltpu.VMEM_SHARED((S, D), jnp.float32),
        ],
    )(data, seg)


def _sc_counts(seg, aux):
    return pl.kernel(
        _cnt_body,
        out_type=jax.ShapeDtypeStruct((NC, S, CW), jnp.float32),
        mesh=_MESH,
        scratch_types=[
            pltpu.VMEM((NB, CH), jnp.int32),
            pltpu.VMEM((TAIL,), jnp.int32),
            pltpu.VMEM((CH, CW), jnp.float32),
            pltpu.VMEM((CH, CW), jnp.float32),
            pltpu.SemaphoreType.DMA((NB,)),
            pltpu.SemaphoreType.DMA((NB,)),
            pltpu.VMEM_SHARED((S, CW), jnp.float32),
        ],
    )(seg, aux)


def _combine_body(acc_ref, cnt_ref, out_ref):
    sums = acc_ref[0] + acc_ref[1]
    counts = cnt_ref[0, :, 0:1] + cnt_ref[1, :, 0:1]
    out_ref[...] = sums / jnp.maximum(counts, 1.0)


@jax.jit
def kernel(data, segment_ids):
    seg = segment_ids.astype(jnp.int32)
    aux = jnp.concatenate([jnp.ones((1, CH, CW), jnp.float32),
                           jnp.zeros((1, CH, CW), jnp.float32)])
    acc = _sc_sums(data, seg)
    cnt = _sc_counts(seg, aux)
    return pl.pallas_call(
        _combine_body,
        out_shape=jax.ShapeDtypeStruct((S, D), jnp.float32),
    )(acc, cnt)
